# baseline (speedup 1.0000x reference)
import functools

import jax
import jax.numpy as jnp
from jax.experimental import pallas as pl

_LN_EPS = 1e-5
_ATTN_EPS = 1e-8


def _dot(a, b, dims):
    return jax.lax.dot_general(a, b, (dims, ((), ())),
                               preferred_element_type=jnp.float32)


def _body(slots_ref, x_ref, wq_ref, wk_ref, wv_ref, wu_ref, wg_ref,
          out_ref, attn_ref, *, inv_sqrt_d):
    x = x_ref[0]
    slots = slots_ref[0]
    N, D = x.shape
    K = slots.shape[0]

    ones8_d = jnp.full((8, D), 1.0 / D, dtype=jnp.float32)
    ones_n8 = jnp.full((N, 8), 1.0, dtype=jnp.float32)

    mu_r = _dot(ones8_d, x, ((1,), (1,)))[:1]
    msq_r = _dot(ones8_d, x * x, ((1,), (1,)))[:1]
    is_r = jax.lax.rsqrt(msq_r - mu_r * mu_r + _LN_EPS)

    sm = jnp.mean(slots, axis=-1, keepdims=True)
    sv = jnp.mean((slots - sm) ** 2, axis=-1, keepdims=True)
    q = _dot((slots - sm) * jax.lax.rsqrt(sv + _LN_EPS),
             wq_ref[...], ((1,), (0,)))

    qk = _dot(q, wk_ref[...], ((1,), (1,)))
    ck = _dot(ones8_d, wk_ref[...], ((1,), (0,))) * D
    qck = _dot(q, ck, ((1,), (1,)))[:, :1]
    raw = _dot(qk, x, ((1,), (1,)))
    logits = (is_r * inv_sqrt_d) * raw - (mu_r * is_r * inv_sqrt_d) * qck

    logits = logits - jnp.max(logits, axis=0, keepdims=True)
    e = jnp.exp(logits)
    attn = e / jnp.sum(e, axis=0, keepdims=True)

    rs = _dot(attn, ones_n8, ((1,), (0,)))[:, :1]
    a_s = (attn / (rs + _ATTN_EPS)) * is_r
    t = _dot(a_s, x, ((1,), (0,)))
    am = _dot(a_s, mu_r, ((1,), (1,)))
    cv = _dot(ones8_d, wv_ref[...], ((1,), (0,)))[:1] * D
    updates = _dot(t, wv_ref[...], ((1,), (0,))) - am * cv

    slots_sa = slots + _dot(updates, wu_ref[...], ((1,), (0,)))

    adj = _dot(attn, attn, ((1,), (1,)))
    adj = adj / (jnp.sum(adj, axis=1, keepdims=True) + _ATTN_EPS)

    agg = _dot(adj, slots_sa, ((1,), (0,)))
    refined = jnp.maximum(_dot(agg, wg_ref[...], ((1,), (0,))), 0.0)

    out_ref[0] = slots_sa + refined
    attn_ref[0] = attn


@jax.jit
def kernel(slots, inputs, Wq, Wk, Wv, Wu, Wg):
    B, K, D = slots.shape
    N = inputs.shape[1] * inputs.shape[2]
    x = inputs.reshape(B, N, D)

    w_spec = pl.BlockSpec((D, D), lambda b: (0, 0))
    out_slots, attn = pl.pallas_call(
        functools.partial(_body, inv_sqrt_d=float(1.0 / (D ** 0.5))),
        grid=(B,),
        in_specs=[
            pl.BlockSpec((1, K, D), lambda b: (b, 0, 0)),
            pl.BlockSpec((1, N, D), lambda b: (b, 0, 0)),
            w_spec, w_spec, w_spec, w_spec, w_spec,
        ],
        out_specs=[
            pl.BlockSpec((1, K, D), lambda b: (b, 0, 0)),
            pl.BlockSpec((1, K, N), lambda b: (b, 0, 0)),
        ],
        out_shape=[
            jax.ShapeDtypeStruct((B, K, D), jnp.float32),
            jax.ShapeDtypeStruct((B, K, N), jnp.float32),
        ],
    )(slots, x, Wq, Wk, Wv, Wu, Wg)
    return out_slots, attn


# speedup vs baseline: 1.1401x; 1.1401x over previous
import functools

import jax
import jax.numpy as jnp
from jax.experimental import pallas as pl

_LN_EPS = 1e-5
_ATTN_EPS = 1e-8


def _dot(a, b, dims):
    return jax.lax.dot_general(a, b, (dims, ((), ())),
                               preferred_element_type=jnp.float32)


def _body(slots_ref, x_ref, wq_ref, wk_ref, wv_ref, wu_ref, wg_ref,
          out_ref, attn_ref, *, inv_sqrt_d):
    for i in range(x_ref.shape[0]):
        _one_frame(slots_ref[i], x_ref[i], wq_ref, wk_ref, wv_ref, wu_ref,
                   wg_ref, out_ref, attn_ref, i, inv_sqrt_d)


def _one_frame(slots, x, wq_ref, wk_ref, wv_ref, wu_ref, wg_ref,
               out_ref, attn_ref, i, inv_sqrt_d):
    N, D = x.shape
    K = slots.shape[0]

    ones8_d = jnp.full((8, D), 1.0 / D, dtype=jnp.float32)
    ones_n8 = jnp.full((N, 8), 1.0, dtype=jnp.float32)

    mu_r = _dot(ones8_d, x, ((1,), (1,)))[:1]
    msq_r = _dot(ones8_d, x * x, ((1,), (1,)))[:1]
    is_r = jax.lax.rsqrt(msq_r - mu_r * mu_r + _LN_EPS)

    sm = jnp.mean(slots, axis=-1, keepdims=True)
    sv = jnp.mean((slots - sm) ** 2, axis=-1, keepdims=True)
    q = _dot((slots - sm) * jax.lax.rsqrt(sv + _LN_EPS),
             wq_ref[...], ((1,), (0,)))

    qk = _dot(q, wk_ref[...], ((1,), (1,)))
    ck = _dot(ones8_d, wk_ref[...], ((1,), (0,))) * D
    qck = _dot(q, ck, ((1,), (1,)))[:, :1]
    raw = _dot(qk, x, ((1,), (1,)))
    logits = (is_r * inv_sqrt_d) * raw - (mu_r * is_r * inv_sqrt_d) * qck

    logits = logits - jnp.max(logits, axis=0, keepdims=True)
    e = jnp.exp(logits)
    attn = e / jnp.sum(e, axis=0, keepdims=True)

    rs = _dot(attn, ones_n8, ((1,), (0,)))[:, :1]
    a_s = (attn / (rs + _ATTN_EPS)) * is_r
    t = _dot(a_s, x, ((1,), (0,)))
    am = _dot(a_s, mu_r, ((1,), (1,)))
    cv = _dot(ones8_d, wv_ref[...], ((1,), (0,)))[:1] * D
    updates = _dot(t, wv_ref[...], ((1,), (0,))) - am * cv

    slots_sa = slots + _dot(updates, wu_ref[...], ((1,), (0,)))

    adj = _dot(attn, attn, ((1,), (1,)))
    adj = adj / (jnp.sum(adj, axis=1, keepdims=True) + _ATTN_EPS)

    agg = _dot(adj, slots_sa, ((1,), (0,)))
    refined = jnp.maximum(_dot(agg, wg_ref[...], ((1,), (0,))), 0.0)

    out_ref[i] = slots_sa + refined
    attn_ref[i] = attn


_BB = 4


@jax.jit
def kernel(slots, inputs, Wq, Wk, Wv, Wu, Wg):
    B, K, D = slots.shape
    N = inputs.shape[1] * inputs.shape[2]
    x = inputs.reshape(B, N, D)

    w_spec = pl.BlockSpec((D, D), lambda b: (0, 0))
    out_slots, attn = pl.pallas_call(
        functools.partial(_body, inv_sqrt_d=float(1.0 / (D ** 0.5))),
        grid=(B // _BB,),
        in_specs=[
            pl.BlockSpec((_BB, K, D), lambda b: (b, 0, 0)),
            pl.BlockSpec((_BB, N, D), lambda b: (b, 0, 0)),
            w_spec, w_spec, w_spec, w_spec, w_spec,
        ],
        out_specs=[
            pl.BlockSpec((_BB, K, D), lambda b: (b, 0, 0)),
            pl.BlockSpec((_BB, K, N), lambda b: (b, 0, 0)),
        ],
        out_shape=[
            jax.ShapeDtypeStruct((B, K, D), jnp.float32),
            jax.ShapeDtypeStruct((B, K, N), jnp.float32),
        ],
    )(slots, x, Wq, Wk, Wv, Wu, Wg)
    return out_slots, attn


# speedup vs baseline: 1.9281x; 1.6912x over previous
import functools

import jax
import jax.numpy as jnp
from jax.experimental import pallas as pl

_LN_EPS = 1e-5
_ATTN_EPS = 1e-8
_BB = 4


def _dot(a, b, dims):
    return jax.lax.dot_general(a, b, (dims, ((), ())),
                               preferred_element_type=jnp.float32)


def _body(slots_ref, x_ref, wq_ref, wk_ref, wv_ref, wu_ref, wg_ref,
          out_ref, attn_ref, *, inv_sqrt_d):
    BB, N, D = x_ref.shape
    K = slots_ref.shape[1]
    wq, wk, wv, wu, wg = (wq_ref[...], wk_ref[...], wv_ref[...],
                          wu_ref[...], wg_ref[...])

    ones8_d = jnp.full((8, D), 1.0 / D, dtype=jnp.float32)
    ones_n8 = jnp.full((N, 8), 1.0, dtype=jnp.float32)

    ck = _dot(ones8_d, wk, ((1,), (0,))) * D
    cv = _dot(ones8_d, wv, ((1,), (0,)))[:1] * D

    s_all = slots_ref[...].reshape(BB * K, D)
    sm = jnp.mean(s_all, axis=-1, keepdims=True)
    sv = jnp.mean((s_all - sm) ** 2, axis=-1, keepdims=True)
    q_all = _dot((s_all - sm) * jax.lax.rsqrt(sv + _LN_EPS), wq, ((1,), (0,)))
    qk_all = _dot(q_all, wk, ((1,), (1,)))
    qck_all = _dot(q_all, ck, ((1,), (1,)))[:, :1]

    x_all = x_ref[...].reshape(BB * N, D)
    mu_all = _dot(ones8_d, x_all, ((1,), (1,)))[:1]
    msq_all = _dot(ones8_d, x_all * x_all, ((1,), (1,)))[:1]
    is_all = jax.lax.rsqrt(msq_all - mu_all * mu_all + _LN_EPS)

    xs, mus, iss, qks, qcks = [], [], [], [], []
    for i in range(BB):
        xs.append(x_ref[i])
        mus.append(mu_all[:, i * N:(i + 1) * N])
        iss.append(is_all[:, i * N:(i + 1) * N])
        qks.append(qk_all[i * K:(i + 1) * K, :])
        qcks.append(qck_all[i * K:(i + 1) * K, :])

    raws = [_dot(qks[i], xs[i], ((1,), (1,))) for i in range(BB)]

    attns = []
    for i in range(BB):
        logits = (iss[i] * inv_sqrt_d) * raws[i] \
            - (mus[i] * iss[i] * inv_sqrt_d) * qcks[i]
        logits = logits - jnp.max(logits, axis=0, keepdims=True)
        e = jnp.exp(logits)
        attns.append(e / jnp.sum(e, axis=0, keepdims=True))

    rss = [_dot(attns[i], ones_n8, ((1,), (0,)))[:, :1] for i in range(BB)]
    a_ss = [(attns[i] / (rss[i] + _ATTN_EPS)) * iss[i] for i in range(BB)]

    ts = [_dot(a_ss[i], xs[i], ((1,), (0,))) for i in range(BB)]
    ams = [_dot(a_ss[i], mus[i], ((1,), (1,))) for i in range(BB)]

    adjs = [_dot(attns[i], attns[i], ((1,), (1,))) for i in range(BB)]
    for i in range(BB):
        updates = _dot(ts[i], wv, ((1,), (0,))) - ams[i] * cv
        slots_sa = slots_ref[i] + _dot(updates, wu, ((1,), (0,)))
        adj = adjs[i] / (jnp.sum(adjs[i], axis=1, keepdims=True) + _ATTN_EPS)
        agg = _dot(adj, slots_sa, ((1,), (0,)))
        refined = jnp.maximum(_dot(agg, wg, ((1,), (0,))), 0.0)
        out_ref[i] = slots_sa + refined
        attn_ref[i] = attns[i]


@jax.jit
def kernel(slots, inputs, Wq, Wk, Wv, Wu, Wg):
    B, K, D = slots.shape
    N = inputs.shape[1] * inputs.shape[2]
    x = inputs.reshape(B, N, D)

    w_spec = pl.BlockSpec((D, D), lambda b: (0, 0))
    out_slots, attn = pl.pallas_call(
        functools.partial(_body, inv_sqrt_d=float(1.0 / (D ** 0.5))),
        grid=(B // _BB,),
        in_specs=[
            pl.BlockSpec((_BB, K, D), lambda b: (b, 0, 0)),
            pl.BlockSpec((_BB, N, D), lambda b: (b, 0, 0)),
            w_spec, w_spec, w_spec, w_spec, w_spec,
        ],
        out_specs=[
            pl.BlockSpec((_BB, K, D), lambda b: (b, 0, 0)),
            pl.BlockSpec((_BB, K, N), lambda b: (b, 0, 0)),
        ],
        out_shape=[
            jax.ShapeDtypeStruct((B, K, D), jnp.float32),
            jax.ShapeDtypeStruct((B, K, N), jnp.float32),
        ],
    )(slots, x, Wq, Wk, Wv, Wu, Wg)
    return out_slots, attn


# speedup vs baseline: 2.1559x; 1.1182x over previous
import functools

import jax
import jax.numpy as jnp
from jax.experimental import pallas as pl

_LN_EPS = 1e-5
_ATTN_EPS = 1e-8
_BB = 8


def _dot(a, b, dims):
    return jax.lax.dot_general(a, b, (dims, ((), ())),
                               preferred_element_type=jnp.float32)


def _body(slots_ref, x_ref, wq_ref, wk_ref, wv_ref, wu_ref, wg_ref,
          out_ref, attn_ref, *, inv_sqrt_d):
    BB, N, D = x_ref.shape
    K = slots_ref.shape[1]
    wq, wk, wv, wu, wg = (wq_ref[...], wk_ref[...], wv_ref[...],
                          wu_ref[...], wg_ref[...])

    ones8_d = jnp.full((8, D), 1.0 / D, dtype=jnp.float32)
    ones_n8 = jnp.full((N, 8), 1.0, dtype=jnp.float32)

    ck = _dot(ones8_d, wk, ((1,), (0,))) * D
    cv = _dot(ones8_d, wv, ((1,), (0,)))[:1] * D

    s_all = slots_ref[...].reshape(BB * K, D)
    sm = jnp.mean(s_all, axis=-1, keepdims=True)
    sv = jnp.mean((s_all - sm) ** 2, axis=-1, keepdims=True)
    q_all = _dot((s_all - sm) * jax.lax.rsqrt(sv + _LN_EPS), wq, ((1,), (0,)))
    qk_all = _dot(q_all, wk, ((1,), (1,)))
    qck_all = _dot(q_all, ck, ((1,), (1,)))[:, :1]

    x_all = x_ref[...].reshape(BB * N, D)
    mu_all = _dot(ones8_d, x_all, ((1,), (1,)))[:1]
    msq_all = _dot(ones8_d, x_all * x_all, ((1,), (1,)))[:1]
    is_all = jax.lax.rsqrt(msq_all - mu_all * mu_all + _LN_EPS)

    xs, mus, iss, qks, qcks = [], [], [], [], []
    for i in range(BB):
        xs.append(x_ref[i])
        mus.append(mu_all[:, i * N:(i + 1) * N])
        iss.append(is_all[:, i * N:(i + 1) * N])
        qks.append(qk_all[i * K:(i + 1) * K, :])
        qcks.append(qck_all[i * K:(i + 1) * K, :])

    raws = [_dot(qks[i], xs[i], ((1,), (1,))) for i in range(BB)]

    attns = []
    for i in range(BB):
        logits = (iss[i] * inv_sqrt_d) * raws[i] \
            - (mus[i] * iss[i] * inv_sqrt_d) * qcks[i]
        logits = logits - jnp.max(logits, axis=0, keepdims=True)
        e = jnp.exp(logits)
        attns.append(e / jnp.sum(e, axis=0, keepdims=True))

    rss = [_dot(attns[i], ones_n8, ((1,), (0,)))[:, :1] for i in range(BB)]
    a_ss = [(attns[i] / (rss[i] + _ATTN_EPS)) * iss[i] for i in range(BB)]

    ts = [_dot(a_ss[i], xs[i], ((1,), (0,))) for i in range(BB)]
    ams = [_dot(a_ss[i], mus[i], ((1,), (1,))) for i in range(BB)]

    adjs = [_dot(attns[i], attns[i], ((1,), (1,))) for i in range(BB)]
    for i in range(BB):
        updates = _dot(ts[i], wv, ((1,), (0,))) - ams[i] * cv
        slots_sa = slots_ref[i] + _dot(updates, wu, ((1,), (0,)))
        adj = adjs[i] / (jnp.sum(adjs[i], axis=1, keepdims=True) + _ATTN_EPS)
        agg = _dot(adj, slots_sa, ((1,), (0,)))
        refined = jnp.maximum(_dot(agg, wg, ((1,), (0,))), 0.0)
        out_ref[i] = slots_sa + refined
        attn_ref[i] = attns[i]


@jax.jit
def kernel(slots, inputs, Wq, Wk, Wv, Wu, Wg):
    B, K, D = slots.shape
    N = inputs.shape[1] * inputs.shape[2]
    x = inputs.reshape(B, N, D)

    w_spec = pl.BlockSpec((D, D), lambda b: (0, 0))
    out_slots, attn = pl.pallas_call(
        functools.partial(_body, inv_sqrt_d=float(1.0 / (D ** 0.5))),
        grid=(B // _BB,),
        in_specs=[
            pl.BlockSpec((_BB, K, D), lambda b: (b, 0, 0)),
            pl.BlockSpec((_BB, N, D), lambda b: (b, 0, 0)),
            w_spec, w_spec, w_spec, w_spec, w_spec,
        ],
        out_specs=[
            pl.BlockSpec((_BB, K, D), lambda b: (b, 0, 0)),
            pl.BlockSpec((_BB, K, N), lambda b: (b, 0, 0)),
        ],
        out_shape=[
            jax.ShapeDtypeStruct((B, K, D), jnp.float32),
            jax.ShapeDtypeStruct((B, K, N), jnp.float32),
        ],
    )(slots, x, Wq, Wk, Wv, Wu, Wg)
    return out_slots, attn


# speedup vs baseline: 2.1996x; 1.0203x over previous
import functools

import jax
import jax.numpy as jnp
from jax.experimental import pallas as pl

_LN_EPS = 1e-5
_ATTN_EPS = 1e-8
_BB = 16


def _dot(a, b, dims):
    return jax.lax.dot_general(a, b, (dims, ((), ())),
                               preferred_element_type=jnp.float32)


def _body(slots_ref, x_ref, wq_ref, wk_ref, wv_ref, wu_ref, wg_ref,
          out_ref, attn_ref, *, inv_sqrt_d):
    BB, N, D = x_ref.shape
    K = slots_ref.shape[1]
    wq, wk, wv, wu, wg = (wq_ref[...], wk_ref[...], wv_ref[...],
                          wu_ref[...], wg_ref[...])

    ones8_d = jnp.full((8, D), 1.0 / D, dtype=jnp.float32)
    ones_n8 = jnp.full((N, 8), 1.0, dtype=jnp.float32)

    ck = _dot(ones8_d, wk, ((1,), (0,))) * D
    cv = _dot(ones8_d, wv, ((1,), (0,)))[:1] * D

    s_all = slots_ref[...].reshape(BB * K, D)
    sm = jnp.mean(s_all, axis=-1, keepdims=True)
    sv = jnp.mean((s_all - sm) ** 2, axis=-1, keepdims=True)
    q_all = _dot((s_all - sm) * jax.lax.rsqrt(sv + _LN_EPS), wq, ((1,), (0,)))
    qk_all = _dot(q_all, wk, ((1,), (1,)))
    qck_all = _dot(q_all, ck, ((1,), (1,)))[:, :1]

    x_all = x_ref[...].reshape(BB * N, D)
    mu_all = _dot(ones8_d, x_all, ((1,), (1,)))[:1]
    msq_all = _dot(ones8_d, x_all * x_all, ((1,), (1,)))[:1]
    is_all = jax.lax.rsqrt(msq_all - mu_all * mu_all + _LN_EPS)

    xs, mus, iss, qks, qcks = [], [], [], [], []
    for i in range(BB):
        xs.append(x_ref[i])
        mus.append(mu_all[:, i * N:(i + 1) * N])
        iss.append(is_all[:, i * N:(i + 1) * N])
        qks.append(qk_all[i * K:(i + 1) * K, :])
        qcks.append(qck_all[i * K:(i + 1) * K, :])

    raws = [_dot(qks[i], xs[i], ((1,), (1,))) for i in range(BB)]

    attns = []
    for i in range(BB):
        logits = (iss[i] * inv_sqrt_d) * raws[i] \
            - (mus[i] * iss[i] * inv_sqrt_d) * qcks[i]
        logits = logits - jnp.max(logits, axis=0, keepdims=True)
        e = jnp.exp(logits)
        attns.append(e / jnp.sum(e, axis=0, keepdims=True))

    rss = [_dot(attns[i], ones_n8, ((1,), (0,)))[:, :1] for i in range(BB)]
    a_ss = [(attns[i] / (rss[i] + _ATTN_EPS)) * iss[i] for i in range(BB)]

    ts = [_dot(a_ss[i], xs[i], ((1,), (0,))) for i in range(BB)]
    ams = [_dot(a_ss[i], mus[i], ((1,), (1,))) for i in range(BB)]

    adjs = [_dot(attns[i], attns[i], ((1,), (1,))) for i in range(BB)]
    for i in range(BB):
        updates = _dot(ts[i], wv, ((1,), (0,))) - ams[i] * cv
        slots_sa = slots_ref[i] + _dot(updates, wu, ((1,), (0,)))
        adj = adjs[i] / (jnp.sum(adjs[i], axis=1, keepdims=True) + _ATTN_EPS)
        agg = _dot(adj, slots_sa, ((1,), (0,)))
        refined = jnp.maximum(_dot(agg, wg, ((1,), (0,))), 0.0)
        out_ref[i] = slots_sa + refined
        attn_ref[i] = attns[i]


@jax.jit
def kernel(slots, inputs, Wq, Wk, Wv, Wu, Wg):
    B, K, D = slots.shape
    N = inputs.shape[1] * inputs.shape[2]
    x = inputs.reshape(B, N, D)

    w_spec = pl.BlockSpec((D, D), lambda b: (0, 0))
    out_slots, attn = pl.pallas_call(
        functools.partial(_body, inv_sqrt_d=float(1.0 / (D ** 0.5))),
        grid=(B // _BB,),
        in_specs=[
            pl.BlockSpec((_BB, K, D), lambda b: (b, 0, 0)),
            pl.BlockSpec((_BB, N, D), lambda b: (b, 0, 0)),
            w_spec, w_spec, w_spec, w_spec, w_spec,
        ],
        out_specs=[
            pl.BlockSpec((_BB, K, D), lambda b: (b, 0, 0)),
            pl.BlockSpec((_BB, K, N), lambda b: (b, 0, 0)),
        ],
        out_shape=[
            jax.ShapeDtypeStruct((B, K, D), jnp.float32),
            jax.ShapeDtypeStruct((B, K, N), jnp.float32),
        ],
    )(slots, x, Wq, Wk, Wv, Wu, Wg)
    return out_slots, attn


# speedup vs baseline: 2.2005x; 1.0004x over previous
import functools

import jax
import jax.numpy as jnp
from jax.experimental import pallas as pl

_LN_EPS = 1e-5
_ATTN_EPS = 1e-8
_BB = 16


def _dot(a, b, dims):
    return jax.lax.dot_general(a, b, (dims, ((), ())),
                               preferred_element_type=jnp.float32)


def _body(slots_ref, x_ref, wq_ref, wk_ref, wv_ref, wu_ref, wg_ref,
          out_ref, attn_ref, *, inv_sqrt_d):
    BB, N, D = x_ref.shape
    K = slots_ref.shape[1]
    wq, wk, wv, wu, wg = (wq_ref[...], wk_ref[...], wv_ref[...],
                          wu_ref[...], wg_ref[...])

    ones8_d = jnp.full((8, D), 1.0 / D, dtype=jnp.float32)
    ones_n8 = jnp.full((N, 8), 1.0, dtype=jnp.float32)

    ck = _dot(ones8_d, wk, ((1,), (0,))) * D
    cv = _dot(ones8_d, wv, ((1,), (0,)))[:1] * D

    s_all = slots_ref[...].reshape(BB * K, D)
    sm = jnp.mean(s_all, axis=-1, keepdims=True)
    sv = jnp.mean((s_all - sm) ** 2, axis=-1, keepdims=True)
    q_all = _dot((s_all - sm) * jax.lax.rsqrt(sv + _LN_EPS), wq, ((1,), (0,)))
    qk_all = _dot(q_all, wk, ((1,), (1,)))
    qck_all = _dot(q_all, ck, ((1,), (1,)))[:, :1]

    x_all = x_ref[...].reshape(BB * N, D)
    mu_all = _dot(ones8_d, x_all, ((1,), (1,)))[:1]
    msq_all = _dot(ones8_d, x_all * x_all, ((1,), (1,)))[:1]
    is_all = jax.lax.rsqrt(msq_all - mu_all * mu_all + _LN_EPS)

    xs, mus, iss, qks, qcks = [], [], [], [], []
    for i in range(BB):
        xs.append(x_ref[i])
        mus.append(mu_all[:, i * N:(i + 1) * N])
        iss.append(is_all[:, i * N:(i + 1) * N])
        qks.append(qk_all[i * K:(i + 1) * K, :])
        qcks.append(qck_all[i * K:(i + 1) * K, :])

    raws = [_dot(qks[i], xs[i], ((1,), (1,))) for i in range(BB)]

    attns, ws = [], []
    for i in range(BB):
        g1 = iss[i] * inv_sqrt_d
        e = jnp.exp((g1 * raws[i]) - (mus[i] * g1) * qcks[i])
        crinv = 1.0 / jnp.sum(e, axis=0, keepdims=True)
        attn = e * crinv
        attns.append(attn)
        ws.append(e * (crinv * iss[i]))
        attn_ref[i] = attn

    rss = [_dot(attns[i], ones_n8, ((1,), (0,)))[:, :1] for i in range(BB)]
    ts = [_dot(ws[i], xs[i], ((1,), (0,))) for i in range(BB)]
    ams = [_dot(ws[i], mus[i], ((1,), (1,))) for i in range(BB)]

    adjs = [_dot(attns[i], attns[i], ((1,), (1,))) for i in range(BB)]
    for i in range(BB):
        rinv = 1.0 / (rss[i] + _ATTN_EPS)
        updates = _dot(ts[i] * rinv, wv, ((1,), (0,))) - (ams[i] * rinv) * cv
        slots_sa = slots_ref[i] + _dot(updates, wu, ((1,), (0,)))
        adj = adjs[i] / (jnp.sum(adjs[i], axis=1, keepdims=True) + _ATTN_EPS)
        agg = _dot(adj, slots_sa, ((1,), (0,)))
        refined = jnp.maximum(_dot(agg, wg, ((1,), (0,))), 0.0)
        out_ref[i] = slots_sa + refined


@jax.jit
def kernel(slots, inputs, Wq, Wk, Wv, Wu, Wg):
    B, K, D = slots.shape
    N = inputs.shape[1] * inputs.shape[2]
    x = inputs.reshape(B, N, D)

    w_spec = pl.BlockSpec((D, D), lambda b: (0, 0))
    out_slots, attn = pl.pallas_call(
        functools.partial(_body, inv_sqrt_d=float(1.0 / (D ** 0.5))),
        grid=(B // _BB,),
        in_specs=[
            pl.BlockSpec((_BB, K, D), lambda b: (b, 0, 0)),
            pl.BlockSpec((_BB, N, D), lambda b: (b, 0, 0)),
            w_spec, w_spec, w_spec, w_spec, w_spec,
        ],
        out_specs=[
            pl.BlockSpec((_BB, K, D), lambda b: (b, 0, 0)),
            pl.BlockSpec((_BB, K, N), lambda b: (b, 0, 0)),
        ],
        out_shape=[
            jax.ShapeDtypeStruct((B, K, D), jnp.float32),
            jax.ShapeDtypeStruct((B, K, N), jnp.float32),
        ],
    )(slots, x, Wq, Wk, Wv, Wu, Wg)
    return out_slots, attn


# speedup vs baseline: 2.4565x; 1.1164x over previous
import functools

import jax
import jax.numpy as jnp
from jax.experimental import pallas as pl

_LN_EPS = 1e-5
_ATTN_EPS = 1e-8
_BB = 16


def _dot(a, b, dims):
    return jax.lax.dot_general(a, b, (dims, ((), ())),
                               preferred_element_type=jnp.float32)


def _body(slots_ref, x_ref, wq_ref, wk_ref, wv_ref, wu_ref, wg_ref,
          out_ref, attn_ref, *, inv_sqrt_d):
    BB, N, D = x_ref.shape
    K = slots_ref.shape[1]
    wq, wk, wv, wu, wg = (wq_ref[...], wk_ref[...], wv_ref[...],
                          wu_ref[...], wg_ref[...])

    ones8_d = jnp.full((8, D), 1.0, dtype=jnp.float32)
    ones_n8 = jnp.full((N, 8), 1.0, dtype=jnp.float32)
    inv_d = 1.0 / D

    ck = _dot(ones8_d, wk, ((1,), (0,)))
    cv = _dot(ones8_d, wv, ((1,), (0,)))[:1]

    s_all = slots_ref[...].reshape(BB * K, D)
    sm = jnp.mean(s_all, axis=-1, keepdims=True)
    sv = jnp.mean((s_all - sm) ** 2, axis=-1, keepdims=True)
    q_all = _dot((s_all - sm) * jax.lax.rsqrt(sv + _LN_EPS), wq, ((1,), (0,)))
    qk_all = _dot(q_all, wk, ((1,), (1,)))
    qck_all = _dot(q_all, ck, ((1,), (1,)))[:, :1]

    x_all = x_ref[...].reshape(BB * N, D)
    xb = x_all.astype(jnp.bfloat16)
    ssq_all = _dot(ones8_d.astype(jnp.bfloat16), xb * xb, ((1,), (1,)))[:1]

    xs, qcks = [], []
    for i in range(BB):
        xs.append(x_ref[i])
        qcks.append(qck_all[i * K:(i + 1) * K, :])

    lefts = [jnp.concatenate([qk_all[i * K:(i + 1) * K, :], ones8_d], axis=0)
             for i in range(BB)]
    raw24s = [_dot(lefts[i], xs[i], ((1,), (1,))) for i in range(BB)]

    raws, mus, iss = [], [], []
    for i in range(BB):
        raws.append(raw24s[i][:K])
        mu = raw24s[i][K:K + 1] * inv_d
        ssq = ssq_all[:, i * N:(i + 1) * N]
        mus.append(mu)
        iss.append(jax.lax.rsqrt(ssq * inv_d - mu * mu + _LN_EPS))

    attns, ws = [], []
    for i in range(BB):
        g1 = iss[i] * inv_sqrt_d
        e = jnp.exp((g1 * raws[i]) - (mus[i] * g1) * qcks[i])
        crinv = 1.0 / jnp.sum(e, axis=0, keepdims=True)
        attn = e * crinv
        attns.append(attn)
        ws.append(e * (crinv * iss[i]))
        attn_ref[i] = attn

    rss = [_dot(attns[i], ones_n8, ((1,), (0,)))[:, :1] for i in range(BB)]
    ts = [_dot(ws[i], xs[i], ((1,), (0,))) for i in range(BB)]
    ams = [_dot(ws[i], mus[i], ((1,), (1,))) for i in range(BB)]

    adjs = [_dot(attns[i], attns[i], ((1,), (1,))) for i in range(BB)]
    for i in range(BB):
        rinv = 1.0 / (rss[i] + _ATTN_EPS)
        updates = _dot(ts[i] * rinv, wv, ((1,), (0,))) - (ams[i] * rinv) * cv
        slots_sa = slots_ref[i] + _dot(updates, wu, ((1,), (0,)))
        adj = adjs[i] / (jnp.sum(adjs[i], axis=1, keepdims=True) + _ATTN_EPS)
        agg = _dot(adj, slots_sa, ((1,), (0,)))
        refined = jnp.maximum(_dot(agg, wg, ((1,), (0,))), 0.0)
        out_ref[i] = slots_sa + refined


@jax.jit
def kernel(slots, inputs, Wq, Wk, Wv, Wu, Wg):
    B, K, D = slots.shape
    N = inputs.shape[1] * inputs.shape[2]
    x = inputs.reshape(B, N, D)

    w_spec = pl.BlockSpec((D, D), lambda b: (0, 0))
    out_slots, attn = pl.pallas_call(
        functools.partial(_body, inv_sqrt_d=float(1.0 / (D ** 0.5))),
        grid=(B // _BB,),
        in_specs=[
            pl.BlockSpec((_BB, K, D), lambda b: (b, 0, 0)),
            pl.BlockSpec((_BB, N, D), lambda b: (b, 0, 0)),
            w_spec, w_spec, w_spec, w_spec, w_spec,
        ],
        out_specs=[
            pl.BlockSpec((_BB, K, D), lambda b: (b, 0, 0)),
            pl.BlockSpec((_BB, K, N), lambda b: (b, 0, 0)),
        ],
        out_shape=[
            jax.ShapeDtypeStruct((B, K, D), jnp.float32),
            jax.ShapeDtypeStruct((B, K, N), jnp.float32),
        ],
    )(slots, x, Wq, Wk, Wv, Wu, Wg)
    return out_slots, attn
